# baseline (speedup 1.0000x reference)
import functools

import jax
import jax.numpy as jnp
from jax import lax
from jax.experimental import pallas as pl
from jax.experimental.pallas import tpu as pltpu
from jax.experimental.pallas import tpu_sc as plsc

_NC = 2
_NS = 16
_NW = _NC * _NS

_F32 = jnp.float32


def _mesh():
    return plsc.VectorSubcoreMesh(
        core_axis_name="c", subcore_axis_name="s",
        num_cores=_NC, num_subcores=_NS)


def _make_gather(E, D):
    CH = 80
    NB = 5
    ew = E // _NW
    nchunk = ew // CH
    ngroup = nchunk // NB

    scratch = ([pltpu.VMEM((CH,), jnp.int32) for _ in range(2 * NB)]
               + [pltpu.VMEM((CH, D), _F32) for _ in range(2 * NB)]
               + [pltpu.SemaphoreType.DMA for _ in range(5 * NB)])

    @functools.partial(
        pl.kernel,
        out_type=jax.ShapeDtypeStruct((E, D), _F32),
        mesh=_mesh(),
        scratch_types=scratch,
    )
    def gather_k(oa_hbm, oc_hbm, sidx_hbm, oidx_hbm, out_hbm, *refs):
        idxs = refs[0:NB]
        idxo = refs[NB:2 * NB]
        rows_s = refs[2 * NB:3 * NB]
        rows_o = refs[3 * NB:4 * NB]
        sems = refs[4 * NB:]
        isem_s = sems[0:NB]
        isem_o = sems[NB:2 * NB]
        gsem_s = sems[2 * NB:3 * NB]
        gsem_o = sems[3 * NB:4 * NB]
        ssem = sems[4 * NB:5 * NB]

        wid = lax.axis_index("s") * _NC + lax.axis_index("c")
        base = wid * ew

        def group(g, carry):
            for b in range(NB):
                off = base + (g * NB + b) * CH
                sl = pl.ds(off, CH)

                @pl.when(g > 0)
                def _(b=b, sl=sl):
                    pltpu.make_async_copy(rows_s[b], out_hbm.at[sl],
                                          ssem[b]).wait()

                pltpu.async_copy(sidx_hbm.at[sl], idxs[b], isem_s[b])
                pltpu.async_copy(oidx_hbm.at[sl], idxo[b], isem_o[b])
            for b in range(NB):
                pltpu.make_async_copy(sidx_hbm.at[pl.ds(0, CH)], idxs[b],
                                      isem_s[b]).wait()
                pltpu.make_async_copy(oidx_hbm.at[pl.ds(0, CH)], idxo[b],
                                      isem_o[b]).wait()
                pltpu.async_copy(oa_hbm.at[idxs[b]], rows_s[b], gsem_s[b])
                pltpu.async_copy(oc_hbm.at[idxo[b]], rows_o[b], gsem_o[b])
            for b in range(NB):
                off = base + (g * NB + b) * CH
                sl = pl.ds(off, CH)
                pltpu.make_async_copy(oa_hbm.at[idxs[b]], rows_s[b],
                                      gsem_s[b]).wait()
                pltpu.make_async_copy(oc_hbm.at[idxo[b]], rows_o[b],
                                      gsem_o[b]).wait()

                def add_row(i, carry, b=b):
                    for j in range(D // 16):
                        cs = pl.ds(j * 16, 16)
                        plsc.addupdate(rows_s[b].at[i, cs],
                                       rows_o[b][i, cs])
                    return carry

                lax.fori_loop(0, CH, add_row, 0)
                pltpu.async_copy(rows_s[b], out_hbm.at[sl], ssem[b])
            return carry

        lax.fori_loop(0, ngroup, group, 0)
        for b in range(NB):
            sl = pl.ds(base + b * CH, CH)
            pltpu.make_async_copy(rows_s[b], out_hbm.at[sl], ssem[b]).wait()

    return gather_k


def _make_scatter(E, N, D):
    CH = 80
    NB = 2
    ew = E // _NW
    nchunk = ew // CH
    ngroup = nchunk // NB
    nrem = nchunk - ngroup * NB
    step = (N // _NS) // 8 * 8
    tail = N - step * _NS

    scratch = ([pltpu.VMEM((CH,), jnp.int32) for _ in range(2 * NB)]
               + [pltpu.VMEM((CH, D), _F32) for _ in range(2 * NB)]
               + [pltpu.VMEM((CH,), _F32), pltpu.VMEM((step,), _F32),
                  pltpu.VMEM_SHARED((N, D), _F32),
                  pltpu.VMEM_SHARED((N,), _F32)]
               + [pltpu.SemaphoreType.DMA for _ in range(8 * NB)])

    @functools.partial(
        pl.kernel,
        out_type=(jax.ShapeDtypeStruct((_NC, N, D), _F32),
                  jax.ShapeDtypeStruct((_NC * N,), _F32)),
        mesh=_mesh(),
        scratch_types=scratch,
    )
    def scatter_k(news_hbm, newo_hbm, sidx_hbm, oidx_hbm, zp_hbm,
                  pooled_out, counts_out, *refs):
        idxs = refs[0:NB]
        idxo = refs[NB:2 * NB]
        rows_s = refs[2 * NB:3 * NB]
        rows_o = refs[3 * NB:4 * NB]
        ones_v, zc_v, pooled_sh, counts_sh = refs[4 * NB:4 * NB + 4]
        sems = refs[4 * NB + 4:]
        isem_s = sems[0:NB]
        isem_o = sems[NB:2 * NB]
        rsem_s = sems[2 * NB:3 * NB]
        rsem_o = sems[3 * NB:4 * NB]
        psem_s = sems[4 * NB:5 * NB]
        psem_o = sems[5 * NB:6 * NB]
        csem_s = sems[6 * NB:7 * NB]
        csem_o = sems[7 * NB:8 * NB]

        cid = lax.axis_index("c")
        sid = lax.axis_index("s")
        wid = sid * _NC + cid
        base = wid * ew
        my = pl.ds(sid * step, step)

        def fill_ones(i, carry):
            ones_v[pl.ds(i * 16, 16)] = jnp.ones((16,), _F32)
            return carry

        lax.fori_loop(0, CH // 16, fill_ones, 0)
        ones_c = ones_v

        def fill_zero(i, carry):
            zc_v[pl.ds(i * 16, 16)] = jnp.zeros((16,), _F32)
            return carry

        lax.fori_loop(0, step // 16, fill_zero, 0)

        pltpu.sync_copy(zp_hbm.at[my], pooled_sh.at[my])
        pltpu.sync_copy(zc_v, counts_sh.at[my])
        if tail:
            tl = pl.ds(step * _NS, tail)

            @pl.when(sid == _NS - 1)
            def _():
                pltpu.sync_copy(zp_hbm.at[tl], pooled_sh.at[tl])
                pltpu.sync_copy(zc_v.at[pl.ds(0, tail)], counts_sh.at[tl])
        plsc.subcore_barrier()

        def group(g, carry):
            for b in range(NB):
                off = base + (g * NB + b) * CH
                sl = pl.ds(off, CH)

                @pl.when(g > 0)
                def _(b=b):
                    pltpu.make_async_copy(rows_s[b], pooled_sh.at[idxs[b]],
                                          psem_s[b]).wait()
                    pltpu.make_async_copy(rows_o[b], pooled_sh.at[idxo[b]],
                                          psem_o[b]).wait()
                    pltpu.make_async_copy(ones_c, counts_sh.at[idxs[b]],
                                          csem_s[b]).wait()
                    pltpu.make_async_copy(ones_c, counts_sh.at[idxo[b]],
                                          csem_o[b]).wait()

                pltpu.async_copy(sidx_hbm.at[sl], idxs[b], isem_s[b])
                pltpu.async_copy(oidx_hbm.at[sl], idxo[b], isem_o[b])
                pltpu.async_copy(news_hbm.at[sl], rows_s[b], rsem_s[b])
                pltpu.async_copy(newo_hbm.at[sl], rows_o[b], rsem_o[b])
            for b in range(NB):
                off = base + (g * NB + b) * CH
                sl = pl.ds(off, CH)
                pltpu.make_async_copy(sidx_hbm.at[sl], idxs[b],
                                      isem_s[b]).wait()
                pltpu.make_async_copy(oidx_hbm.at[sl], idxo[b],
                                      isem_o[b]).wait()
                pltpu.make_async_copy(news_hbm.at[sl], rows_s[b],
                                      rsem_s[b]).wait()
                pltpu.make_async_copy(newo_hbm.at[sl], rows_o[b],
                                      rsem_o[b]).wait()
                pltpu.async_copy(rows_s[b], pooled_sh.at[idxs[b]], psem_s[b],
                                 add=True)
                pltpu.async_copy(rows_o[b], pooled_sh.at[idxo[b]], psem_o[b],
                                 add=True)
                pltpu.async_copy(ones_c, counts_sh.at[idxs[b]], csem_s[b],
                                 add=True)
                pltpu.async_copy(ones_c, counts_sh.at[idxo[b]], csem_o[b],
                                 add=True)
            return carry

        lax.fori_loop(0, ngroup, group, 0)
        for b in range(NB):
            pltpu.make_async_copy(rows_s[b], pooled_sh.at[idxs[b]],
                                  psem_s[b]).wait()
            pltpu.make_async_copy(rows_o[b], pooled_sh.at[idxo[b]],
                                  psem_o[b]).wait()
            pltpu.make_async_copy(ones_c, counts_sh.at[idxs[b]],
                                  csem_s[b]).wait()
            pltpu.make_async_copy(ones_c, counts_sh.at[idxo[b]],
                                  csem_o[b]).wait()
        for q in range(nrem):
            sl = pl.ds(base + (ngroup * NB + q) * CH, CH)
            pltpu.sync_copy(sidx_hbm.at[sl], idxs[q])
            pltpu.sync_copy(oidx_hbm.at[sl], idxo[q])
            pltpu.sync_copy(news_hbm.at[sl], rows_s[q])
            pltpu.sync_copy(newo_hbm.at[sl], rows_o[q])
            pltpu.sync_copy(rows_s[q], pooled_sh.at[idxs[q]], add=True)
            pltpu.sync_copy(rows_o[q], pooled_sh.at[idxo[q]], add=True)
            pltpu.sync_copy(ones_c, counts_sh.at[idxs[q]], add=True)
            pltpu.sync_copy(ones_c, counts_sh.at[idxo[q]], add=True)
        plsc.subcore_barrier()

        pltpu.sync_copy(pooled_sh.at[my], pooled_out.at[cid, my])
        pltpu.sync_copy(counts_sh.at[my], zc_v)
        pltpu.sync_copy(zc_v, counts_out.at[pl.ds(cid * N + sid * step, step)])
        if tail:
            tl2 = pl.ds(step * _NS, tail)

            @pl.when(sid == _NS - 1)
            def _():
                pltpu.sync_copy(pooled_sh.at[tl2], pooled_out.at[cid, tl2])
                pltpu.sync_copy(counts_sh.at[tl2], zc_v.at[pl.ds(0, tail)])
                pltpu.sync_copy(zc_v.at[pl.ds(0, tail)],
                                counts_out.at[pl.ds(cid * N + step * _NS,
                                                    tail)])

    return scatter_k


def _premix_body(obj_ref, m1_ref, bp_ref, oa_ref, oc_ref, r_ref):
    d = oa_ref.shape[1]
    t = jnp.dot(obj_ref[...], m1_ref[...], preferred_element_type=_F32)
    oa_ref[...] = t[:, :d]
    oc_ref[...] = t[:, d:2 * d]
    r_ref[...] = t[:, 2 * d:] + bp_ref[...]


def _edge_body(pre_ref, p_ref, w1ap_ref, b1a_ref, w1b_ref, b1b_ref,
               wpp_ref, bpp_ref, ns_ref, np_ref, no_ref):
    h = b1a_ref.shape[1]
    dout = np_ref.shape[1]
    p = p_ref[...]
    pm = jnp.dot(p, w1ap_ref[...], preferred_element_type=_F32)
    h1 = jnp.maximum(pre_ref[...] + pm + b1a_ref[...], 0.0)
    t = jnp.dot(h1, w1b_ref[...], preferred_element_type=_F32) + b1b_ref[...]
    t = jnp.maximum(t, 0.0)
    ns_ref[...] = t[:, :h]
    np_ref[...] = (t[:, h:h + dout]
                   + jnp.dot(p, wpp_ref[...], preferred_element_type=_F32)
                   + bpp_ref[...])
    no_ref[...] = t[:, h + dout:]


def _node_body(pp_ref, cc_ref, r_ref, w2a_ref, b2a_ref, w2b_ref, b2b_ref,
               out_ref):
    pooled = pp_ref[0] + pp_ref[1]
    cnt = cc_ref[0] + cc_ref[1]
    cnt = jnp.maximum(cnt, 1.0)
    pooled = pooled / cnt
    h2 = jnp.maximum(
        jnp.dot(pooled, w2a_ref[...], preferred_element_type=_F32)
        + b2a_ref[...], 0.0)
    out_ref[...] = (jnp.maximum(
        jnp.dot(h2, w2b_ref[...], preferred_element_type=_F32)
        + b2b_ref[...], 0.0) + r_ref[...])


def kernel(obj_vecs, pred_vecs, edges, W1a, b1a, W1b, b1b, W2a, b2a, W2b,
           b2b, Wp, bp, Wpp, bpp):
    N, D = obj_vecs.shape
    E = pred_vecs.shape[0]
    H = W1a.shape[1]
    DOUT = W2b.shape[1]

    s_idx = edges[:, 0]
    o_idx = edges[:, 1]

    m1 = jnp.concatenate([W1a[:D], W1a[2 * D:], Wp], axis=1)
    oa, oc, r = pl.pallas_call(
        _premix_body,
        out_shape=(jax.ShapeDtypeStruct((N, H), _F32),
                   jax.ShapeDtypeStruct((N, H), _F32),
                   jax.ShapeDtypeStruct((N, DOUT), _F32)),
    )(obj_vecs, m1, bp.reshape(1, DOUT))

    pre = _make_gather(E, H)(oa, oc, s_idx, o_idx)

    BE = 2000
    grid = (E // BE,)
    row_spec = pl.BlockSpec((BE, H), lambda i: (i, 0))
    full = lambda a, b: pl.BlockSpec((a, b), lambda i: (0, 0))
    new_s, new_p, new_o = pl.pallas_call(
        _edge_body,
        grid=grid,
        in_specs=[row_spec, pl.BlockSpec((BE, D), lambda i: (i, 0)),
                  full(D, H), full(1, H), full(H, 2 * H + DOUT),
                  full(1, 2 * H + DOUT), full(D, DOUT), full(1, DOUT)],
        out_specs=[row_spec, pl.BlockSpec((BE, DOUT), lambda i: (i, 0)),
                   row_spec],
        out_shape=(jax.ShapeDtypeStruct((E, H), _F32),
                   jax.ShapeDtypeStruct((E, DOUT), _F32),
                   jax.ShapeDtypeStruct((E, H), _F32)),
        compiler_params=pltpu.CompilerParams(
            dimension_semantics=("parallel",)),
    )(pre, pred_vecs, W1a[D:2 * D], b1a.reshape(1, H), W1b,
      b1b.reshape(1, 2 * H + DOUT), Wpp, bpp.reshape(1, DOUT))

    zp = jnp.zeros((N, H), _F32)
    pooled_parts, counts_flat = _make_scatter(E, N, H)(
        new_s, new_o, s_idx, o_idx, zp)
    counts_parts = counts_flat.reshape(2, N, 1)

    new_obj = pl.pallas_call(
        _node_body,
        out_shape=jax.ShapeDtypeStruct((N, DOUT), _F32),
    )(pooled_parts, counts_parts, r, W2a, b2a.reshape(1, H), W2b,
      b2b.reshape(1, DOUT))

    return (new_obj, new_p)


# speedup vs baseline: 1.0461x; 1.0461x over previous
import functools

import jax
import jax.numpy as jnp
from jax import lax
from jax.experimental import pallas as pl
from jax.experimental.pallas import tpu as pltpu
from jax.experimental.pallas import tpu_sc as plsc

_NC = 2
_NS = 16
_NW = _NC * _NS

_F32 = jnp.float32


def _mesh():
    return plsc.VectorSubcoreMesh(
        core_axis_name="c", subcore_axis_name="s",
        num_cores=_NC, num_subcores=_NS)


def _make_gather(E, D):
    CH = 80
    NB = 5
    ew = E // _NW
    nchunk = ew // CH
    ngroup = nchunk // NB

    scratch = ([pltpu.VMEM((CH,), jnp.int32) for _ in range(2 * NB)]
               + [pltpu.VMEM((CH, D), _F32) for _ in range(2 * NB)]
               + [pltpu.SemaphoreType.DMA for _ in range(5 * NB)])

    @functools.partial(
        pl.kernel,
        out_type=jax.ShapeDtypeStruct((E, D), _F32),
        mesh=_mesh(),
        scratch_types=scratch,
    )
    def gather_k(oa_hbm, oc_hbm, sidx_hbm, oidx_hbm, out_hbm, *refs):
        idxs = refs[0:NB]
        idxo = refs[NB:2 * NB]
        rows_s = refs[2 * NB:3 * NB]
        rows_o = refs[3 * NB:4 * NB]
        sems = refs[4 * NB:]
        isem_s = sems[0:NB]
        isem_o = sems[NB:2 * NB]
        gsem_s = sems[2 * NB:3 * NB]
        gsem_o = sems[3 * NB:4 * NB]
        ssem = sems[4 * NB:5 * NB]

        wid = lax.axis_index("s") * _NC + lax.axis_index("c")
        base = wid * ew

        def group(g, carry):
            for b in range(NB):
                off = base + (g * NB + b) * CH
                sl = pl.ds(off, CH)

                @pl.when(g > 0)
                def _(b=b, sl=sl):
                    pltpu.make_async_copy(rows_s[b], out_hbm.at[sl],
                                          ssem[b]).wait()

                pltpu.async_copy(sidx_hbm.at[sl], idxs[b], isem_s[b])
                pltpu.async_copy(oidx_hbm.at[sl], idxo[b], isem_o[b])
            for b in range(NB):
                pltpu.make_async_copy(sidx_hbm.at[pl.ds(0, CH)], idxs[b],
                                      isem_s[b]).wait()
                pltpu.make_async_copy(oidx_hbm.at[pl.ds(0, CH)], idxo[b],
                                      isem_o[b]).wait()
                pltpu.async_copy(oa_hbm.at[idxs[b]], rows_s[b], gsem_s[b])
                pltpu.async_copy(oc_hbm.at[idxo[b]], rows_o[b], gsem_o[b])
            for b in range(NB):
                off = base + (g * NB + b) * CH
                sl = pl.ds(off, CH)
                pltpu.make_async_copy(oa_hbm.at[idxs[b]], rows_s[b],
                                      gsem_s[b]).wait()
                pltpu.make_async_copy(oc_hbm.at[idxo[b]], rows_o[b],
                                      gsem_o[b]).wait()

                def add_row(i, carry, b=b):
                    for j in range(D // 16):
                        cs = pl.ds(j * 16, 16)
                        plsc.addupdate(rows_s[b].at[i, cs],
                                       rows_o[b][i, cs])
                    return carry

                lax.fori_loop(0, CH, add_row, 0)
                pltpu.async_copy(rows_s[b], out_hbm.at[sl], ssem[b])
            return carry

        lax.fori_loop(0, ngroup, group, 0)
        for b in range(NB):
            sl = pl.ds(base + b * CH, CH)
            pltpu.make_async_copy(rows_s[b], out_hbm.at[sl], ssem[b]).wait()

    return gather_k


def _make_scatter(E, N, D):
    CH = 40
    NB = 3
    ew = E // _NW
    nchunk = ew // CH
    ngroup = nchunk // NB
    nrem = nchunk - ngroup * NB
    step = (N // _NS) // 8 * 8
    tail = N - step * _NS

    scratch = ([pltpu.VMEM((CH,), jnp.int32) for _ in range(2 * NB)]
               + [pltpu.VMEM((CH, D), _F32) for _ in range(2 * NB)]
               + [pltpu.VMEM((48,), _F32), pltpu.VMEM((step,), _F32),
                  pltpu.VMEM_SHARED((N, D), _F32),
                  pltpu.VMEM_SHARED((N,), _F32)]
               + [pltpu.SemaphoreType.DMA for _ in range(8 * NB)])

    @functools.partial(
        pl.kernel,
        out_type=(jax.ShapeDtypeStruct((_NC, N, D), _F32),
                  jax.ShapeDtypeStruct((_NC * N,), _F32)),
        mesh=_mesh(),
        scratch_types=scratch,
    )
    def scatter_k(news_hbm, newo_hbm, sidx_hbm, oidx_hbm, zp_hbm,
                  pooled_out, counts_out, *refs):
        idxs = refs[0:NB]
        idxo = refs[NB:2 * NB]
        rows_s = refs[2 * NB:3 * NB]
        rows_o = refs[3 * NB:4 * NB]
        ones_v, zc_v, pooled_sh, counts_sh = refs[4 * NB:4 * NB + 4]
        sems = refs[4 * NB + 4:]
        isem_s = sems[0:NB]
        isem_o = sems[NB:2 * NB]
        rsem_s = sems[2 * NB:3 * NB]
        rsem_o = sems[3 * NB:4 * NB]
        psem_s = sems[4 * NB:5 * NB]
        psem_o = sems[5 * NB:6 * NB]
        csem_s = sems[6 * NB:7 * NB]
        csem_o = sems[7 * NB:8 * NB]

        cid = lax.axis_index("c")
        sid = lax.axis_index("s")
        wid = sid * _NC + cid
        base = wid * ew
        my = pl.ds(sid * step, step)

        def fill_ones(i, carry):
            ones_v[pl.ds(i * 16, 16)] = jnp.ones((16,), _F32)
            return carry

        lax.fori_loop(0, 3, fill_ones, 0)
        ones_c = ones_v.at[pl.ds(0, CH)]

        def fill_zero(i, carry):
            zc_v[pl.ds(i * 16, 16)] = jnp.zeros((16,), _F32)
            return carry

        lax.fori_loop(0, step // 16, fill_zero, 0)

        pltpu.sync_copy(zp_hbm.at[my], pooled_sh.at[my])
        pltpu.sync_copy(zc_v, counts_sh.at[my])
        if tail:
            tl = pl.ds(step * _NS, tail)

            @pl.when(sid == _NS - 1)
            def _():
                pltpu.sync_copy(zp_hbm.at[tl], pooled_sh.at[tl])
                pltpu.sync_copy(zc_v.at[pl.ds(0, tail)], counts_sh.at[tl])
        plsc.subcore_barrier()

        def group(g, carry):
            for b in range(NB):
                off = base + (g * NB + b) * CH
                sl = pl.ds(off, CH)

                @pl.when(g > 0)
                def _(b=b):
                    pltpu.make_async_copy(rows_s[b], pooled_sh.at[idxs[b]],
                                          psem_s[b]).wait()
                    pltpu.make_async_copy(rows_o[b], pooled_sh.at[idxo[b]],
                                          psem_o[b]).wait()
                    pltpu.make_async_copy(ones_c, counts_sh.at[idxs[b]],
                                          csem_s[b]).wait()
                    pltpu.make_async_copy(ones_c, counts_sh.at[idxo[b]],
                                          csem_o[b]).wait()

                pltpu.async_copy(sidx_hbm.at[sl], idxs[b], isem_s[b])
                pltpu.async_copy(oidx_hbm.at[sl], idxo[b], isem_o[b])
                pltpu.async_copy(news_hbm.at[sl], rows_s[b], rsem_s[b])
                pltpu.async_copy(newo_hbm.at[sl], rows_o[b], rsem_o[b])
            for b in range(NB):
                off = base + (g * NB + b) * CH
                sl = pl.ds(off, CH)
                pltpu.make_async_copy(sidx_hbm.at[sl], idxs[b],
                                      isem_s[b]).wait()
                pltpu.make_async_copy(oidx_hbm.at[sl], idxo[b],
                                      isem_o[b]).wait()
                pltpu.make_async_copy(news_hbm.at[sl], rows_s[b],
                                      rsem_s[b]).wait()
                pltpu.make_async_copy(newo_hbm.at[sl], rows_o[b],
                                      rsem_o[b]).wait()
                pltpu.async_copy(rows_s[b], pooled_sh.at[idxs[b]], psem_s[b],
                                 add=True)
                pltpu.async_copy(rows_o[b], pooled_sh.at[idxo[b]], psem_o[b],
                                 add=True)
                pltpu.async_copy(ones_c, counts_sh.at[idxs[b]], csem_s[b],
                                 add=True)
                pltpu.async_copy(ones_c, counts_sh.at[idxo[b]], csem_o[b],
                                 add=True)
            return carry

        lax.fori_loop(0, ngroup, group, 0)
        for b in range(NB):
            pltpu.make_async_copy(rows_s[b], pooled_sh.at[idxs[b]],
                                  psem_s[b]).wait()
            pltpu.make_async_copy(rows_o[b], pooled_sh.at[idxo[b]],
                                  psem_o[b]).wait()
            pltpu.make_async_copy(ones_c, counts_sh.at[idxs[b]],
                                  csem_s[b]).wait()
            pltpu.make_async_copy(ones_c, counts_sh.at[idxo[b]],
                                  csem_o[b]).wait()
        for q in range(nrem):
            sl = pl.ds(base + (ngroup * NB + q) * CH, CH)
            pltpu.sync_copy(sidx_hbm.at[sl], idxs[q])
            pltpu.sync_copy(oidx_hbm.at[sl], idxo[q])
            pltpu.sync_copy(news_hbm.at[sl], rows_s[q])
            pltpu.sync_copy(newo_hbm.at[sl], rows_o[q])
            pltpu.sync_copy(rows_s[q], pooled_sh.at[idxs[q]], add=True)
            pltpu.sync_copy(rows_o[q], pooled_sh.at[idxo[q]], add=True)
            pltpu.sync_copy(ones_c, counts_sh.at[idxs[q]], add=True)
            pltpu.sync_copy(ones_c, counts_sh.at[idxo[q]], add=True)
        plsc.subcore_barrier()

        pltpu.sync_copy(pooled_sh.at[my], pooled_out.at[cid, my])
        pltpu.sync_copy(counts_sh.at[my], zc_v)
        pltpu.sync_copy(zc_v, counts_out.at[pl.ds(cid * N + sid * step, step)])
        if tail:
            tl2 = pl.ds(step * _NS, tail)

            @pl.when(sid == _NS - 1)
            def _():
                pltpu.sync_copy(pooled_sh.at[tl2], pooled_out.at[cid, tl2])
                pltpu.sync_copy(counts_sh.at[tl2], zc_v.at[pl.ds(0, tail)])
                pltpu.sync_copy(zc_v.at[pl.ds(0, tail)],
                                counts_out.at[pl.ds(cid * N + step * _NS,
                                                    tail)])

    return scatter_k


def _premix_body(obj_ref, m1_ref, bp_ref, oa_ref, oc_ref, r_ref):
    d = oa_ref.shape[1]
    t = jnp.dot(obj_ref[...], m1_ref[...], preferred_element_type=_F32)
    oa_ref[...] = t[:, :d]
    oc_ref[...] = t[:, d:2 * d]
    r_ref[...] = t[:, 2 * d:] + bp_ref[...]


def _edge_body(pre_ref, p_ref, w1ap_ref, b1a_ref, w1b_ref, b1b_ref,
               wpp_ref, bpp_ref, ns_ref, np_ref, no_ref):
    h = b1a_ref.shape[1]
    dout = np_ref.shape[1]
    p = p_ref[...]
    pm = jnp.dot(p, w1ap_ref[...], preferred_element_type=_F32)
    h1 = jnp.maximum(pre_ref[...] + pm + b1a_ref[...], 0.0)
    t = jnp.dot(h1, w1b_ref[...], preferred_element_type=_F32) + b1b_ref[...]
    t = jnp.maximum(t, 0.0)
    ns_ref[...] = t[:, :h]
    np_ref[...] = (t[:, h:h + dout]
                   + jnp.dot(p, wpp_ref[...], preferred_element_type=_F32)
                   + bpp_ref[...])
    no_ref[...] = t[:, h + dout:]


def _node_body(pp_ref, cc_ref, r_ref, w2a_ref, b2a_ref, w2b_ref, b2b_ref,
               out_ref):
    pooled = pp_ref[0] + pp_ref[1]
    cnt = cc_ref[0] + cc_ref[1]
    cnt = jnp.maximum(cnt, 1.0)
    pooled = pooled / cnt
    h2 = jnp.maximum(
        jnp.dot(pooled, w2a_ref[...], preferred_element_type=_F32)
        + b2a_ref[...], 0.0)
    out_ref[...] = (jnp.maximum(
        jnp.dot(h2, w2b_ref[...], preferred_element_type=_F32)
        + b2b_ref[...], 0.0) + r_ref[...])


def kernel(obj_vecs, pred_vecs, edges, W1a, b1a, W1b, b1b, W2a, b2a, W2b,
           b2b, Wp, bp, Wpp, bpp):
    N, D = obj_vecs.shape
    E = pred_vecs.shape[0]
    H = W1a.shape[1]
    DOUT = W2b.shape[1]

    s_idx = edges[:, 0]
    o_idx = edges[:, 1]

    m1 = jnp.concatenate([W1a[:D], W1a[2 * D:], Wp], axis=1)
    oa, oc, r = pl.pallas_call(
        _premix_body,
        out_shape=(jax.ShapeDtypeStruct((N, H), _F32),
                   jax.ShapeDtypeStruct((N, H), _F32),
                   jax.ShapeDtypeStruct((N, DOUT), _F32)),
    )(obj_vecs, m1, bp.reshape(1, DOUT))

    pre = _make_gather(E, H)(oa, oc, s_idx, o_idx)

    BE = 4000
    grid = (E // BE,)
    row_spec = pl.BlockSpec((BE, H), lambda i: (i, 0))
    full = lambda a, b: pl.BlockSpec((a, b), lambda i: (0, 0))
    new_s, new_p, new_o = pl.pallas_call(
        _edge_body,
        grid=grid,
        in_specs=[row_spec, pl.BlockSpec((BE, D), lambda i: (i, 0)),
                  full(D, H), full(1, H), full(H, 2 * H + DOUT),
                  full(1, 2 * H + DOUT), full(D, DOUT), full(1, DOUT)],
        out_specs=[row_spec, pl.BlockSpec((BE, DOUT), lambda i: (i, 0)),
                   row_spec],
        out_shape=(jax.ShapeDtypeStruct((E, H), _F32),
                   jax.ShapeDtypeStruct((E, DOUT), _F32),
                   jax.ShapeDtypeStruct((E, H), _F32)),
        compiler_params=pltpu.CompilerParams(
            dimension_semantics=("parallel",)),
    )(pre, pred_vecs, W1a[D:2 * D], b1a.reshape(1, H), W1b,
      b1b.reshape(1, 2 * H + DOUT), Wpp, bpp.reshape(1, DOUT))

    zp = jnp.zeros((N, H), _F32)
    pooled_parts, counts_flat = _make_scatter(E, N, H)(
        new_s, new_o, s_idx, o_idx, zp)
    counts_parts = counts_flat.reshape(2, N, 1)

    new_obj = pl.pallas_call(
        _node_body,
        out_shape=jax.ShapeDtypeStruct((N, DOUT), _F32),
    )(pooled_parts, counts_parts, r, W2a, b2a.reshape(1, H), W2b,
      b2b.reshape(1, DOUT))

    return (new_obj, new_p)


# speedup vs baseline: 1.0573x; 1.0107x over previous
import functools

import jax
import jax.numpy as jnp
from jax import lax
from jax.experimental import pallas as pl
from jax.experimental.pallas import tpu as pltpu
from jax.experimental.pallas import tpu_sc as plsc

_NC = 2
_NS = 16
_NW = _NC * _NS

_F32 = jnp.float32


def _mesh():
    return plsc.VectorSubcoreMesh(
        core_axis_name="c", subcore_axis_name="s",
        num_cores=_NC, num_subcores=_NS)


def _make_gather(E, D):
    CH = 80
    NB = 5
    ew = E // _NW
    nchunk = ew // CH
    ngroup = nchunk // NB

    scratch = ([pltpu.VMEM((CH,), jnp.int32) for _ in range(2 * NB)]
               + [pltpu.VMEM((CH, D), _F32) for _ in range(2 * NB)]
               + [pltpu.SemaphoreType.DMA for _ in range(5 * NB)])

    @functools.partial(
        pl.kernel,
        out_type=jax.ShapeDtypeStruct((E, D), _F32),
        mesh=_mesh(),
        scratch_types=scratch,
    )
    def gather_k(oa_hbm, oc_hbm, sidx_hbm, oidx_hbm, out_hbm, *refs):
        idxs = refs[0:NB]
        idxo = refs[NB:2 * NB]
        rows_s = refs[2 * NB:3 * NB]
        rows_o = refs[3 * NB:4 * NB]
        sems = refs[4 * NB:]
        isem_s = sems[0:NB]
        isem_o = sems[NB:2 * NB]
        gsem_s = sems[2 * NB:3 * NB]
        gsem_o = sems[3 * NB:4 * NB]
        ssem = sems[4 * NB:5 * NB]

        wid = lax.axis_index("s") * _NC + lax.axis_index("c")
        base = wid * ew

        def group(g, carry):
            for b in range(NB):
                off = base + (g * NB + b) * CH
                sl = pl.ds(off, CH)

                @pl.when(g > 0)
                def _(b=b, sl=sl):
                    pltpu.make_async_copy(rows_s[b], out_hbm.at[sl],
                                          ssem[b]).wait()

                pltpu.async_copy(sidx_hbm.at[sl], idxs[b], isem_s[b])
                pltpu.async_copy(oidx_hbm.at[sl], idxo[b], isem_o[b])
            for b in range(NB):
                pltpu.make_async_copy(sidx_hbm.at[pl.ds(0, CH)], idxs[b],
                                      isem_s[b]).wait()
                pltpu.make_async_copy(oidx_hbm.at[pl.ds(0, CH)], idxo[b],
                                      isem_o[b]).wait()
                pltpu.async_copy(oa_hbm.at[idxs[b]], rows_s[b], gsem_s[b])
                pltpu.async_copy(oc_hbm.at[idxo[b]], rows_o[b], gsem_o[b])
            for b in range(NB):
                off = base + (g * NB + b) * CH
                sl = pl.ds(off, CH)
                pltpu.make_async_copy(oa_hbm.at[idxs[b]], rows_s[b],
                                      gsem_s[b]).wait()
                pltpu.make_async_copy(oc_hbm.at[idxo[b]], rows_o[b],
                                      gsem_o[b]).wait()

                def add_row(i, carry, b=b):
                    for j in range(D // 16):
                        cs = pl.ds(j * 16, 16)
                        plsc.addupdate(rows_s[b].at[i, cs],
                                       rows_o[b][i, cs])
                    return carry

                lax.fori_loop(0, CH, add_row, 0)
                pltpu.async_copy(rows_s[b], out_hbm.at[sl], ssem[b])
            return carry

        lax.fori_loop(0, ngroup, group, 0)
        for b in range(NB):
            sl = pl.ds(base + b * CH, CH)
            pltpu.make_async_copy(rows_s[b], out_hbm.at[sl], ssem[b]).wait()

    return gather_k


def _make_scatter(E, N, D):
    CH = 40
    NB = 3
    ew = E // _NW
    nchunk = ew // CH
    ngroup = nchunk // NB
    nrem = nchunk - ngroup * NB
    step = (N // _NS) // 8 * 8
    tail = N - step * _NS

    scratch = ([pltpu.VMEM((CH,), jnp.int32) for _ in range(2 * NB)]
               + [pltpu.VMEM((CH, D), _F32) for _ in range(2 * NB)]
               + [pltpu.VMEM((48,), _F32), pltpu.VMEM((step,), _F32),
                  pltpu.VMEM_SHARED((N, D), _F32),
                  pltpu.VMEM_SHARED((N,), _F32)]
               + [pltpu.SemaphoreType.DMA for _ in range(8 * NB)])

    @functools.partial(
        pl.kernel,
        out_type=(jax.ShapeDtypeStruct((_NC, N, D), _F32),
                  jax.ShapeDtypeStruct((_NC * N,), _F32)),
        mesh=_mesh(),
        scratch_types=scratch,
    )
    def scatter_k(news_hbm, newo_hbm, sidx_hbm, oidx_hbm, zp_hbm,
                  pooled_out, counts_out, *refs):
        idxs = refs[0:NB]
        idxo = refs[NB:2 * NB]
        rows_s = refs[2 * NB:3 * NB]
        rows_o = refs[3 * NB:4 * NB]
        ones_v, zc_v, pooled_sh, counts_sh = refs[4 * NB:4 * NB + 4]
        sems = refs[4 * NB + 4:]
        isem_s = sems[0:NB]
        isem_o = sems[NB:2 * NB]
        rsem_s = sems[2 * NB:3 * NB]
        rsem_o = sems[3 * NB:4 * NB]
        psem_s = sems[4 * NB:5 * NB]
        psem_o = sems[5 * NB:6 * NB]
        csem_s = sems[6 * NB:7 * NB]
        csem_o = sems[7 * NB:8 * NB]

        cid = lax.axis_index("c")
        sid = lax.axis_index("s")
        wid = sid * _NC + cid
        base = wid * ew
        my = pl.ds(sid * step, step)

        def fill_ones(i, carry):
            ones_v[pl.ds(i * 16, 16)] = jnp.ones((16,), _F32)
            return carry

        lax.fori_loop(0, 3, fill_ones, 0)
        ones_c = ones_v.at[pl.ds(0, CH)]

        def fill_zero(i, carry):
            zc_v[pl.ds(i * 16, 16)] = jnp.zeros((16,), _F32)
            return carry

        lax.fori_loop(0, step // 16, fill_zero, 0)

        pltpu.sync_copy(zp_hbm.at[my], pooled_sh.at[my])
        pltpu.sync_copy(zc_v, counts_sh.at[my])
        if tail:
            tl = pl.ds(step * _NS, tail)

            @pl.when(sid == _NS - 1)
            def _():
                pltpu.sync_copy(zp_hbm.at[tl], pooled_sh.at[tl])
                pltpu.sync_copy(zc_v.at[pl.ds(0, tail)], counts_sh.at[tl])
        plsc.subcore_barrier()

        def group(g, carry):
            for b in range(NB):
                off = base + (g * NB + b) * CH
                sl = pl.ds(off, CH)

                @pl.when(g > 0)
                def _(b=b):
                    pltpu.make_async_copy(rows_s[b], pooled_sh.at[idxs[b]],
                                          psem_s[b]).wait()
                    pltpu.make_async_copy(rows_o[b], pooled_sh.at[idxo[b]],
                                          psem_o[b]).wait()
                    pltpu.make_async_copy(ones_c, counts_sh.at[idxs[b]],
                                          csem_s[b]).wait()
                    pltpu.make_async_copy(ones_c, counts_sh.at[idxo[b]],
                                          csem_o[b]).wait()

                pltpu.async_copy(sidx_hbm.at[sl], idxs[b], isem_s[b])
                pltpu.async_copy(oidx_hbm.at[sl], idxo[b], isem_o[b])
                pltpu.async_copy(news_hbm.at[sl], rows_s[b], rsem_s[b])
                pltpu.async_copy(newo_hbm.at[sl], rows_o[b], rsem_o[b])
            for b in range(NB):
                off = base + (g * NB + b) * CH
                sl = pl.ds(off, CH)
                pltpu.make_async_copy(sidx_hbm.at[sl], idxs[b],
                                      isem_s[b]).wait()
                pltpu.make_async_copy(oidx_hbm.at[sl], idxo[b],
                                      isem_o[b]).wait()
                pltpu.make_async_copy(news_hbm.at[sl], rows_s[b],
                                      rsem_s[b]).wait()
                pltpu.make_async_copy(newo_hbm.at[sl], rows_o[b],
                                      rsem_o[b]).wait()
                pltpu.async_copy(rows_s[b], pooled_sh.at[idxs[b]], psem_s[b],
                                 add=True)
                pltpu.async_copy(rows_o[b], pooled_sh.at[idxo[b]], psem_o[b],
                                 add=True)
                pltpu.async_copy(ones_c, counts_sh.at[idxs[b]], csem_s[b],
                                 add=True)
                pltpu.async_copy(ones_c, counts_sh.at[idxo[b]], csem_o[b],
                                 add=True)
            return carry

        lax.fori_loop(0, ngroup, group, 0)
        for b in range(NB):
            pltpu.make_async_copy(rows_s[b], pooled_sh.at[idxs[b]],
                                  psem_s[b]).wait()
            pltpu.make_async_copy(rows_o[b], pooled_sh.at[idxo[b]],
                                  psem_o[b]).wait()
            pltpu.make_async_copy(ones_c, counts_sh.at[idxs[b]],
                                  csem_s[b]).wait()
            pltpu.make_async_copy(ones_c, counts_sh.at[idxo[b]],
                                  csem_o[b]).wait()
        for q in range(nrem):
            sl = pl.ds(base + (ngroup * NB + q) * CH, CH)
            pltpu.sync_copy(sidx_hbm.at[sl], idxs[q])
            pltpu.sync_copy(oidx_hbm.at[sl], idxo[q])
            pltpu.sync_copy(news_hbm.at[sl], rows_s[q])
            pltpu.sync_copy(newo_hbm.at[sl], rows_o[q])
            pltpu.sync_copy(rows_s[q], pooled_sh.at[idxs[q]], add=True)
            pltpu.sync_copy(rows_o[q], pooled_sh.at[idxo[q]], add=True)
            pltpu.sync_copy(ones_c, counts_sh.at[idxs[q]], add=True)
            pltpu.sync_copy(ones_c, counts_sh.at[idxo[q]], add=True)
        plsc.subcore_barrier()

        pltpu.sync_copy(pooled_sh.at[my], pooled_out.at[cid, my])
        pltpu.sync_copy(counts_sh.at[my], zc_v)
        pltpu.sync_copy(zc_v, counts_out.at[pl.ds(cid * N + sid * step, step)])
        if tail:
            tl2 = pl.ds(step * _NS, tail)

            @pl.when(sid == _NS - 1)
            def _():
                pltpu.sync_copy(pooled_sh.at[tl2], pooled_out.at[cid, tl2])
                pltpu.sync_copy(counts_sh.at[tl2], zc_v.at[pl.ds(0, tail)])
                pltpu.sync_copy(zc_v.at[pl.ds(0, tail)],
                                counts_out.at[pl.ds(cid * N + step * _NS,
                                                    tail)])

    return scatter_k


def _premix_body(obj_ref, m1_ref, bp_ref, oa_ref, oc_ref, r_ref):
    d = oa_ref.shape[1]
    t = jnp.dot(obj_ref[...], m1_ref[...], preferred_element_type=_F32)
    oa_ref[...] = t[:, :d]
    oc_ref[...] = t[:, d:2 * d]
    r_ref[...] = t[:, 2 * d:] + bp_ref[...]


def _edge_body(pre_ref, p_ref, w1ap_ref, b1a_ref, w1b_ref, b1b_ref,
               wpp_ref, bpp_ref, ns_ref, np_ref, no_ref):
    h = b1a_ref.shape[1]
    dout = np_ref.shape[1]
    p = p_ref[...]
    pm = jnp.dot(p, w1ap_ref[...], preferred_element_type=_F32)
    h1 = jnp.maximum(pre_ref[...] + pm + b1a_ref[...], 0.0)
    t = jnp.dot(h1, w1b_ref[...], preferred_element_type=_F32) + b1b_ref[...]
    t = jnp.maximum(t, 0.0)
    ns_ref[...] = t[:, :h]
    np_ref[...] = (t[:, h:h + dout]
                   + jnp.dot(p, wpp_ref[...], preferred_element_type=_F32)
                   + bpp_ref[...])
    no_ref[...] = t[:, h + dout:]


def _node_body(pp_ref, cc_ref, r_ref, w2a_ref, b2a_ref, w2b_ref, b2b_ref,
               out_ref):
    pooled = pp_ref[0] + pp_ref[1]
    cnt = cc_ref[0] + cc_ref[1]
    cnt = jnp.maximum(cnt, 1.0)
    pooled = pooled / cnt
    h2 = jnp.maximum(
        jnp.dot(pooled, w2a_ref[...], preferred_element_type=_F32)
        + b2a_ref[...], 0.0)
    out_ref[...] = (jnp.maximum(
        jnp.dot(h2, w2b_ref[...], preferred_element_type=_F32)
        + b2b_ref[...], 0.0) + r_ref[...])


def kernel(obj_vecs, pred_vecs, edges, W1a, b1a, W1b, b1b, W2a, b2a, W2b,
           b2b, Wp, bp, Wpp, bpp):
    N, D = obj_vecs.shape
    E = pred_vecs.shape[0]
    H = W1a.shape[1]
    DOUT = W2b.shape[1]

    s_idx = edges[:, 0]
    o_idx = edges[:, 1]

    m1 = jnp.concatenate([W1a[:D], W1a[2 * D:], Wp], axis=1)
    oa, oc, r = pl.pallas_call(
        _premix_body,
        out_shape=(jax.ShapeDtypeStruct((N, H), _F32),
                   jax.ShapeDtypeStruct((N, H), _F32),
                   jax.ShapeDtypeStruct((N, DOUT), _F32)),
    )(obj_vecs, m1, bp.reshape(1, DOUT))

    pre = _make_gather(E, H)(oa, oc, s_idx, o_idx)

    BE = 8000
    grid = (E // BE,)
    row_spec = pl.BlockSpec((BE, H), lambda i: (i, 0))
    full = lambda a, b: pl.BlockSpec((a, b), lambda i: (0, 0))
    new_s, new_p, new_o = pl.pallas_call(
        _edge_body,
        grid=grid,
        in_specs=[row_spec, pl.BlockSpec((BE, D), lambda i: (i, 0)),
                  full(D, H), full(1, H), full(H, 2 * H + DOUT),
                  full(1, 2 * H + DOUT), full(D, DOUT), full(1, DOUT)],
        out_specs=[row_spec, pl.BlockSpec((BE, DOUT), lambda i: (i, 0)),
                   row_spec],
        out_shape=(jax.ShapeDtypeStruct((E, H), _F32),
                   jax.ShapeDtypeStruct((E, DOUT), _F32),
                   jax.ShapeDtypeStruct((E, H), _F32)),
        compiler_params=pltpu.CompilerParams(
            dimension_semantics=("parallel",)),
    )(pre, pred_vecs, W1a[D:2 * D], b1a.reshape(1, H), W1b,
      b1b.reshape(1, 2 * H + DOUT), Wpp, bpp.reshape(1, DOUT))

    zp = jnp.zeros((N, H), _F32)
    pooled_parts, counts_flat = _make_scatter(E, N, H)(
        new_s, new_o, s_idx, o_idx, zp)
    counts_parts = counts_flat.reshape(2, N, 1)

    new_obj = pl.pallas_call(
        _node_body,
        out_shape=jax.ShapeDtypeStruct((N, DOUT), _F32),
    )(pooled_parts, counts_parts, r, W2a, b2a.reshape(1, H), W2b,
      b2b.reshape(1, DOUT))

    return (new_obj, new_p)


# speedup vs baseline: 1.0603x; 1.0028x over previous
import functools

import jax
import jax.numpy as jnp
from jax import lax
from jax.experimental import pallas as pl
from jax.experimental.pallas import tpu as pltpu
from jax.experimental.pallas import tpu_sc as plsc

_NC = 2
_NS = 16
_NW = _NC * _NS

_F32 = jnp.float32


def _mesh():
    return plsc.VectorSubcoreMesh(
        core_axis_name="c", subcore_axis_name="s",
        num_cores=_NC, num_subcores=_NS)


def _make_gather(E, D):
    CH = 80
    NB = 5
    ew = E // _NW
    nchunk = ew // CH
    ngroup = nchunk // NB

    scratch = ([pltpu.VMEM((CH,), jnp.int32) for _ in range(2 * NB)]
               + [pltpu.VMEM((CH, D), _F32) for _ in range(2 * NB)]
               + [pltpu.SemaphoreType.DMA for _ in range(5 * NB)])

    @functools.partial(
        pl.kernel,
        out_type=jax.ShapeDtypeStruct((E, D), _F32),
        mesh=_mesh(),
        scratch_types=scratch,
    )
    def gather_k(oa_hbm, oc_hbm, sidx_hbm, oidx_hbm, out_hbm, *refs):
        idxs = refs[0:NB]
        idxo = refs[NB:2 * NB]
        rows_s = refs[2 * NB:3 * NB]
        rows_o = refs[3 * NB:4 * NB]
        sems = refs[4 * NB:]
        isem_s = sems[0:NB]
        isem_o = sems[NB:2 * NB]
        gsem_s = sems[2 * NB:3 * NB]
        gsem_o = sems[3 * NB:4 * NB]
        ssem = sems[4 * NB:5 * NB]

        wid = lax.axis_index("s") * _NC + lax.axis_index("c")
        base = wid * ew

        def group(g, carry):
            for b in range(NB):
                off = base + (g * NB + b) * CH
                sl = pl.ds(off, CH)

                @pl.when(g > 0)
                def _(b=b, sl=sl):
                    pltpu.make_async_copy(rows_s[b], out_hbm.at[sl],
                                          ssem[b]).wait()

                pltpu.async_copy(sidx_hbm.at[sl], idxs[b], isem_s[b])
                pltpu.async_copy(oidx_hbm.at[sl], idxo[b], isem_o[b])
            for b in range(NB):
                pltpu.make_async_copy(sidx_hbm.at[pl.ds(0, CH)], idxs[b],
                                      isem_s[b]).wait()
                pltpu.make_async_copy(oidx_hbm.at[pl.ds(0, CH)], idxo[b],
                                      isem_o[b]).wait()
                pltpu.async_copy(oa_hbm.at[idxs[b]], rows_s[b], gsem_s[b])
                pltpu.async_copy(oc_hbm.at[idxo[b]], rows_o[b], gsem_o[b])
            for b in range(NB):
                off = base + (g * NB + b) * CH
                sl = pl.ds(off, CH)
                pltpu.make_async_copy(oa_hbm.at[idxs[b]], rows_s[b],
                                      gsem_s[b]).wait()
                pltpu.make_async_copy(oc_hbm.at[idxo[b]], rows_o[b],
                                      gsem_o[b]).wait()

                def add_row(i, carry, b=b):
                    for j in range(D // 16):
                        cs = pl.ds(j * 16, 16)
                        plsc.addupdate(rows_s[b].at[i, cs],
                                       rows_o[b][i, cs])
                    return carry

                lax.fori_loop(0, CH, add_row, 0)
                pltpu.async_copy(rows_s[b], out_hbm.at[sl], ssem[b])
            return carry

        lax.fori_loop(0, ngroup, group, 0)
        for b in range(NB):
            sl = pl.ds(base + b * CH, CH)
            pltpu.make_async_copy(rows_s[b], out_hbm.at[sl], ssem[b]).wait()

    return gather_k


def _make_scatter(E, N, D):
    CH = 40
    NB = 3
    ew = E // _NW
    nchunk = ew // CH
    ngroup = nchunk // NB
    nrem = nchunk - ngroup * NB
    step = (N // _NS) // 8 * 8
    tail = N - step * _NS

    scratch = ([pltpu.VMEM((CH,), jnp.int32) for _ in range(2 * NB)]
               + [pltpu.VMEM((CH, D), _F32) for _ in range(2 * NB)]
               + [pltpu.VMEM((48,), _F32), pltpu.VMEM((step,), _F32),
                  pltpu.VMEM_SHARED((N, D), _F32),
                  pltpu.VMEM_SHARED((N,), _F32)]
               + [pltpu.SemaphoreType.DMA for _ in range(8 * NB)])

    @functools.partial(
        pl.kernel,
        out_type=(jax.ShapeDtypeStruct((_NC, N, D), _F32),
                  jax.ShapeDtypeStruct((_NC * N,), _F32)),
        mesh=_mesh(),
        scratch_types=scratch,
    )
    def scatter_k(news_hbm, newo_hbm, sidx_hbm, oidx_hbm, zp_hbm,
                  pooled_out, counts_out, *refs):
        idxs = refs[0:NB]
        idxo = refs[NB:2 * NB]
        rows_s = refs[2 * NB:3 * NB]
        rows_o = refs[3 * NB:4 * NB]
        ones_v, zc_v, pooled_sh, counts_sh = refs[4 * NB:4 * NB + 4]
        sems = refs[4 * NB + 4:]
        isem_s = sems[0:NB]
        isem_o = sems[NB:2 * NB]
        rsem_s = sems[2 * NB:3 * NB]
        rsem_o = sems[3 * NB:4 * NB]
        psem_s = sems[4 * NB:5 * NB]
        psem_o = sems[5 * NB:6 * NB]
        csem_s = sems[6 * NB:7 * NB]
        csem_o = sems[7 * NB:8 * NB]

        cid = lax.axis_index("c")
        sid = lax.axis_index("s")
        wid = sid * _NC + cid
        base = wid * ew
        my = pl.ds(sid * step, step)

        def fill_ones(i, carry):
            ones_v[pl.ds(i * 16, 16)] = jnp.ones((16,), _F32)
            return carry

        lax.fori_loop(0, 3, fill_ones, 0)
        ones_c = ones_v.at[pl.ds(0, CH)]

        def fill_zero(i, carry):
            zc_v[pl.ds(i * 16, 16)] = jnp.zeros((16,), _F32)
            return carry

        lax.fori_loop(0, step // 16, fill_zero, 0)

        pltpu.sync_copy(zp_hbm.at[my], pooled_sh.at[my])
        pltpu.sync_copy(zc_v, counts_sh.at[my])
        if tail:
            tl = pl.ds(step * _NS, tail)

            @pl.when(sid == _NS - 1)
            def _():
                pltpu.sync_copy(zp_hbm.at[tl], pooled_sh.at[tl])
                pltpu.sync_copy(zc_v.at[pl.ds(0, tail)], counts_sh.at[tl])
        plsc.subcore_barrier()

        def group(g, carry):
            for b in range(NB):
                off = base + (g * NB + b) * CH
                sl = pl.ds(off, CH)

                @pl.when(g > 0)
                def _(b=b):
                    pltpu.make_async_copy(rows_s[b], pooled_sh.at[idxs[b]],
                                          psem_s[b]).wait()
                    pltpu.make_async_copy(rows_o[b], pooled_sh.at[idxo[b]],
                                          psem_o[b]).wait()
                    pltpu.make_async_copy(ones_c, counts_sh.at[idxs[b]],
                                          csem_s[b]).wait()
                    pltpu.make_async_copy(ones_c, counts_sh.at[idxo[b]],
                                          csem_o[b]).wait()

                pltpu.async_copy(sidx_hbm.at[sl], idxs[b], isem_s[b])
                pltpu.async_copy(oidx_hbm.at[sl], idxo[b], isem_o[b])
                pltpu.async_copy(news_hbm.at[sl], rows_s[b], rsem_s[b])
                pltpu.async_copy(newo_hbm.at[sl], rows_o[b], rsem_o[b])
            for b in range(NB):
                off = base + (g * NB + b) * CH
                sl = pl.ds(off, CH)
                pltpu.make_async_copy(sidx_hbm.at[sl], idxs[b],
                                      isem_s[b]).wait()
                pltpu.make_async_copy(oidx_hbm.at[sl], idxo[b],
                                      isem_o[b]).wait()
                pltpu.make_async_copy(news_hbm.at[sl], rows_s[b],
                                      rsem_s[b]).wait()
                pltpu.make_async_copy(newo_hbm.at[sl], rows_o[b],
                                      rsem_o[b]).wait()
                pltpu.async_copy(rows_s[b], pooled_sh.at[idxs[b]], psem_s[b],
                                 add=True)
                pltpu.async_copy(rows_o[b], pooled_sh.at[idxo[b]], psem_o[b],
                                 add=True)
                pltpu.async_copy(ones_c, counts_sh.at[idxs[b]], csem_s[b],
                                 add=True)
                pltpu.async_copy(ones_c, counts_sh.at[idxo[b]], csem_o[b],
                                 add=True)
            return carry

        lax.fori_loop(0, ngroup, group, 0)
        for b in range(NB):
            pltpu.make_async_copy(rows_s[b], pooled_sh.at[idxs[b]],
                                  psem_s[b]).wait()
            pltpu.make_async_copy(rows_o[b], pooled_sh.at[idxo[b]],
                                  psem_o[b]).wait()
            pltpu.make_async_copy(ones_c, counts_sh.at[idxs[b]],
                                  csem_s[b]).wait()
            pltpu.make_async_copy(ones_c, counts_sh.at[idxo[b]],
                                  csem_o[b]).wait()
        for q in range(nrem):
            sl = pl.ds(base + (ngroup * NB + q) * CH, CH)
            pltpu.sync_copy(sidx_hbm.at[sl], idxs[q])
            pltpu.sync_copy(oidx_hbm.at[sl], idxo[q])
            pltpu.sync_copy(news_hbm.at[sl], rows_s[q])
            pltpu.sync_copy(newo_hbm.at[sl], rows_o[q])
            pltpu.sync_copy(rows_s[q], pooled_sh.at[idxs[q]], add=True)
            pltpu.sync_copy(rows_o[q], pooled_sh.at[idxo[q]], add=True)
            pltpu.sync_copy(ones_c, counts_sh.at[idxs[q]], add=True)
            pltpu.sync_copy(ones_c, counts_sh.at[idxo[q]], add=True)
        plsc.subcore_barrier()

        pltpu.sync_copy(pooled_sh.at[my], pooled_out.at[cid, my])
        pltpu.sync_copy(counts_sh.at[my], zc_v)
        pltpu.sync_copy(zc_v, counts_out.at[pl.ds(cid * N + sid * step, step)])
        if tail:
            tl2 = pl.ds(step * _NS, tail)

            @pl.when(sid == _NS - 1)
            def _():
                pltpu.sync_copy(pooled_sh.at[tl2], pooled_out.at[cid, tl2])
                pltpu.sync_copy(counts_sh.at[tl2], zc_v.at[pl.ds(0, tail)])
                pltpu.sync_copy(zc_v.at[pl.ds(0, tail)],
                                counts_out.at[pl.ds(cid * N + step * _NS,
                                                    tail)])

    return scatter_k


def _premix_body(obj_ref, m1_ref, bp_ref, oa_ref, oc_ref, r_ref):
    d = oa_ref.shape[1]
    t = jnp.dot(obj_ref[...], m1_ref[...], preferred_element_type=_F32)
    oa_ref[...] = t[:, :d]
    oc_ref[...] = t[:, d:2 * d]
    r_ref[...] = t[:, 2 * d:] + bp_ref[...]


def _edge_body(pre_ref, p_ref, w1ap_ref, b1a_ref, w1b_ref, b1b_ref,
               wpp_ref, bpp_ref, ns_ref, np_ref, no_ref):
    h = b1a_ref.shape[1]
    dout = np_ref.shape[1]
    p = p_ref[...]
    pm = jnp.dot(p, w1ap_ref[...], preferred_element_type=_F32)
    h1 = jnp.maximum(pre_ref[...] + pm + b1a_ref[...], 0.0)
    t = jnp.dot(h1, w1b_ref[...], preferred_element_type=_F32) + b1b_ref[...]
    t = jnp.maximum(t, 0.0)
    ns_ref[...] = t[:, :h]
    np_ref[...] = (t[:, h:h + dout]
                   + jnp.dot(p, wpp_ref[...], preferred_element_type=_F32)
                   + bpp_ref[...])
    no_ref[...] = t[:, h + dout:]


def _node_body(pp_ref, cc_ref, r_ref, w2a_ref, b2a_ref, w2b_ref, b2b_ref,
               out_ref):
    pooled = pp_ref[0] + pp_ref[1]
    cnt = cc_ref[0] + cc_ref[1]
    cnt = jnp.maximum(cnt, 1.0)
    pooled = pooled / cnt
    h2 = jnp.maximum(
        jnp.dot(pooled, w2a_ref[...], preferred_element_type=_F32)
        + b2a_ref[...], 0.0)
    out_ref[...] = (jnp.maximum(
        jnp.dot(h2, w2b_ref[...], preferred_element_type=_F32)
        + b2b_ref[...], 0.0) + r_ref[...])


def kernel(obj_vecs, pred_vecs, edges, W1a, b1a, W1b, b1b, W2a, b2a, W2b,
           b2b, Wp, bp, Wpp, bpp):
    N, D = obj_vecs.shape
    E = pred_vecs.shape[0]
    H = W1a.shape[1]
    DOUT = W2b.shape[1]

    s_idx = edges[:, 0]
    o_idx = edges[:, 1]

    m1 = jnp.concatenate([W1a[:D], W1a[2 * D:], Wp], axis=1)
    oa, oc, r = pl.pallas_call(
        _premix_body,
        out_shape=(jax.ShapeDtypeStruct((N, H), _F32),
                   jax.ShapeDtypeStruct((N, H), _F32),
                   jax.ShapeDtypeStruct((N, DOUT), _F32)),
    )(obj_vecs, m1, bp.reshape(1, DOUT))

    pre = _make_gather(E, H)(oa, oc, s_idx, o_idx)

    BE = 10000
    grid = (E // BE,)
    row_spec = pl.BlockSpec((BE, H), lambda i: (i, 0))
    full = lambda a, b: pl.BlockSpec((a, b), lambda i: (0, 0))
    new_s, new_p, new_o = pl.pallas_call(
        _edge_body,
        grid=grid,
        in_specs=[row_spec, pl.BlockSpec((BE, D), lambda i: (i, 0)),
                  full(D, H), full(1, H), full(H, 2 * H + DOUT),
                  full(1, 2 * H + DOUT), full(D, DOUT), full(1, DOUT)],
        out_specs=[row_spec, pl.BlockSpec((BE, DOUT), lambda i: (i, 0)),
                   row_spec],
        out_shape=(jax.ShapeDtypeStruct((E, H), _F32),
                   jax.ShapeDtypeStruct((E, DOUT), _F32),
                   jax.ShapeDtypeStruct((E, H), _F32)),
        compiler_params=pltpu.CompilerParams(
            dimension_semantics=("parallel",)),
    )(pre, pred_vecs, W1a[D:2 * D], b1a.reshape(1, H), W1b,
      b1b.reshape(1, 2 * H + DOUT), Wpp, bpp.reshape(1, DOUT))

    zp = jnp.zeros((N, H), _F32)
    pooled_parts, counts_flat = _make_scatter(E, N, H)(
        new_s, new_o, s_idx, o_idx, zp)
    counts_parts = counts_flat.reshape(2, N, 1)

    new_obj = pl.pallas_call(
        _node_body,
        out_shape=jax.ShapeDtypeStruct((N, DOUT), _F32),
    )(pooled_parts, counts_parts, r, W2a, b2a.reshape(1, H), W2b,
      b2b.reshape(1, DOUT))

    return (new_obj, new_p)
